# baseline (device time: 361424 ns/iter reference)
import functools

import jax
import jax.numpy as jnp
from jax import lax
from jax.experimental import pallas as pl
from jax.experimental.pallas import tpu as pltpu

N_Z = 4
MB = 256


def kernel(O, Wo):
    B, S, Hs, D = O.shape
    K = Hs * D
    N = Wo.shape[1]
    Sc = S // N_Z
    M = B * Sc
    NSUB = M // MB

    Ob = O.reshape(B, S, K).astype(jnp.bfloat16)
    Oc = Ob.reshape(B, N_Z, Sc, K).transpose(1, 0, 2, 3).reshape(N_Z, M, K)
    Wb = Wo.astype(jnp.bfloat16)

    def body(oc_ref, w_ref, out_ref, comm_ref, p_ref, stage_ref,
             send_sem, recv_sems, copy_sem):
        mx = lax.axis_index("x")
        my = lax.axis_index("y")
        mz = lax.axis_index("z")
        left = (mz - 1) % N_Z
        right = (mz + 1) % N_Z

        barrier = pltpu.get_barrier_semaphore()
        for nz in (left, right):
            pl.semaphore_signal(
                barrier, inc=1,
                device_id=(mx, my, nz),
                device_id_type=pl.DeviceIdType.MESH,
            )
        pl.semaphore_wait(barrier, 2)

        def partial_into(dst, c):
            for i in range(NSUB):
                r = pl.ds(i * MB, MB)
                x = oc_ref[c, r, :]
                y = jnp.dot(x, w_ref[...], preferred_element_type=jnp.float32)
                dst[r, :] = y.astype(jnp.bfloat16)

        partial_into(out_ref, (mz - 1) % N_Z)
        for s in range(N_Z - 1):
            rdma = pltpu.make_async_remote_copy(
                src_ref=out_ref,
                dst_ref=comm_ref.at[s],
                send_sem=send_sem,
                recv_sem=recv_sems.at[s],
                device_id=(mx, my, right),
                device_id_type=pl.DeviceIdType.MESH,
            )
            rdma.start()
            partial_into(p_ref, (mz - 2 - s) % N_Z)
            rdma.wait()
            cp = pltpu.make_async_copy(comm_ref.at[s], stage_ref, copy_sem)
            cp.start()
            cp.wait()
            out_ref[...] = p_ref[...] + stage_ref[...]

        @functools.partial(pl.run_scoped, sem2=pltpu.SemaphoreType.REGULAR)
        def _(sem2):
            for nz in (left, right):
                pl.semaphore_signal(
                    sem2, inc=1,
                    device_id=(mx, my, nz),
                    device_id_type=pl.DeviceIdType.MESH,
                )
            pl.semaphore_wait(sem2, 2)

    out, _ = pl.pallas_call(
        body,
        out_shape=[
            jax.ShapeDtypeStruct((M, N), jnp.bfloat16),
            jax.ShapeDtypeStruct((N_Z - 1, M, N), jnp.bfloat16),
        ],
        in_specs=[
            pl.BlockSpec(memory_space=pltpu.VMEM),
            pl.BlockSpec(memory_space=pltpu.VMEM),
        ],
        out_specs=[
            pl.BlockSpec(memory_space=pltpu.VMEM),
            pl.BlockSpec(memory_space=pl.ANY),
        ],
        scratch_shapes=[
            pltpu.VMEM((M, N), jnp.bfloat16),
            pltpu.VMEM((M, N), jnp.bfloat16),
            pltpu.SemaphoreType.DMA,
            pltpu.SemaphoreType.DMA((N_Z - 1,)),
            pltpu.SemaphoreType.DMA,
        ],
        compiler_params=pltpu.CompilerParams(
            collective_id=0,
            vmem_limit_bytes=100 * 1024 * 1024,
        ),
    )(Oc, Wb)
    return out.reshape(B, Sc, N)


# device time: 239695 ns/iter; 1.5078x vs baseline; 1.5078x over previous
import functools

import jax
import jax.numpy as jnp
from jax import lax
from jax.experimental import pallas as pl
from jax.experimental.pallas import tpu as pltpu

N_Z = 4
MB = 256


def kernel(O, Wo):
    B, S, Hs, D = O.shape
    K = Hs * D
    N = Wo.shape[1]
    Sc = S // N_Z
    M = B * Sc
    H = M // 2
    NSUB = H // MB

    mx_out = lax.axis_index("x")
    Ob = O.reshape(B, S, K).astype(jnp.bfloat16)
    Omy = lax.dynamic_index_in_dim(Ob, mx_out, axis=0, keepdims=False)
    Wb = Wo.astype(jnp.bfloat16)

    def body(o_ref, w_ref, out_ref, comm_ref, p_ref, stage_ref,
             send_sem, recv_sems, x_send_sem, x_recv_sem, copy_sem):
        mx = lax.axis_index("x")
        my = lax.axis_index("y")
        mz = lax.axis_index("z")
        left = (mz - 1) % N_Z
        right = (mz + 1) % N_Z
        row0 = mx * H

        barrier = pltpu.get_barrier_semaphore()
        for dev in ((mx, my, left), (mx, my, right), (1 - mx, my, mz)):
            pl.semaphore_signal(
                barrier, inc=1,
                device_id=dev, device_id_type=pl.DeviceIdType.MESH,
            )
        pl.semaphore_wait(barrier, 3)

        def partial_into(dst, dst_row0, c):
            for i in range(NSUB):
                x = o_ref[pl.ds(c * Sc + i * MB, MB), :]
                y = jnp.dot(x, w_ref[...], preferred_element_type=jnp.float32)
                dst[pl.ds(dst_row0 + i * MB, MB), :] = y.astype(jnp.bfloat16)

        partial_into(out_ref, row0, (mz - 1) % N_Z)
        for s in range(N_Z - 1):
            rdma = pltpu.make_async_remote_copy(
                src_ref=out_ref.at[pl.ds(row0, H)],
                dst_ref=comm_ref.at[s],
                send_sem=send_sem,
                recv_sem=recv_sems.at[s],
                device_id=(mx, my, right),
                device_id_type=pl.DeviceIdType.MESH,
            )
            rdma.start()
            partial_into(p_ref, 0, (mz - 2 - s) % N_Z)
            rdma.wait()
            cp = pltpu.make_async_copy(comm_ref.at[s], stage_ref, copy_sem)
            cp.start()
            cp.wait()
            out_ref[pl.ds(row0, H), :] = p_ref[...] + stage_ref[...]

        xchg = pltpu.make_async_remote_copy(
            src_ref=out_ref.at[pl.ds(row0, H)],
            dst_ref=out_ref.at[pl.ds(row0, H)],
            send_sem=x_send_sem,
            recv_sem=x_recv_sem,
            device_id=(1 - mx, my, mz),
            device_id_type=pl.DeviceIdType.MESH,
        )
        xchg.start()
        xchg.wait()

        @functools.partial(pl.run_scoped, sem2=pltpu.SemaphoreType.REGULAR)
        def _(sem2):
            for dev in ((mx, my, left), (mx, my, right), (1 - mx, my, mz)):
                pl.semaphore_signal(
                    sem2, inc=1,
                    device_id=dev, device_id_type=pl.DeviceIdType.MESH,
                )
            pl.semaphore_wait(sem2, 3)

    out, _ = pl.pallas_call(
        body,
        out_shape=[
            jax.ShapeDtypeStruct((M, N), jnp.bfloat16),
            jax.ShapeDtypeStruct((N_Z - 1, H, N), jnp.bfloat16),
        ],
        in_specs=[
            pl.BlockSpec(memory_space=pltpu.VMEM),
            pl.BlockSpec(memory_space=pltpu.VMEM),
        ],
        out_specs=[
            pl.BlockSpec(memory_space=pltpu.VMEM),
            pl.BlockSpec(memory_space=pl.ANY),
        ],
        scratch_shapes=[
            pltpu.VMEM((H, N), jnp.bfloat16),
            pltpu.VMEM((H, N), jnp.bfloat16),
            pltpu.SemaphoreType.DMA,
            pltpu.SemaphoreType.DMA((N_Z - 1,)),
            pltpu.SemaphoreType.DMA,
            pltpu.SemaphoreType.DMA,
            pltpu.SemaphoreType.DMA,
        ],
        compiler_params=pltpu.CompilerParams(
            collective_id=0,
            vmem_limit_bytes=100 * 1024 * 1024,
        ),
    )(Omy, Wb)
    return out.reshape(B, Sc, N)


# device time: 128428 ns/iter; 2.8142x vs baseline; 1.8664x over previous
import functools

import jax
import jax.numpy as jnp
from jax import lax
from jax.experimental import pallas as pl
from jax.experimental.pallas import tpu as pltpu

N_Z = 4
N_Y = 4


def kernel(O, Wo):
    B, S, Hs, D = O.shape
    K = Hs * D
    N = Wo.shape[1]
    Sc = S // N_Z
    M = B * Sc
    PH = Sc // N_Y

    mx_o = lax.axis_index("x")
    my_o = lax.axis_index("y")
    Of = O.reshape(B, S, K)
    Ob = lax.dynamic_index_in_dim(Of, mx_o, axis=0, keepdims=False)
    Op = lax.dynamic_index_in_dim(
        Ob.reshape(N_Z, N_Y, PH, K), my_o, axis=1, keepdims=False
    ).astype(jnp.bfloat16)
    Wb = Wo.astype(jnp.bfloat16)

    def body(o_ref, w_ref, out_ref, comm_ref, p_ref,
             ring_send, ring_recv, yr_send, yr_recv, yl_send, yl_recv,
             xo_send, xo_recv, xr_send, xr_recv, xl_send, xl_recv):
        mx = lax.axis_index("x")
        my = lax.axis_index("y")
        mz = lax.axis_index("z")
        left = (mz - 1) % N_Z
        right = (mz + 1) % N_Z
        row0 = mx * (M // 2)
        prow = row0 + my * PH

        has_yl = my >= 1
        has_yr = my <= N_Y - 2

        barrier = pltpu.get_barrier_semaphore()
        for dev in ((mx, my, left), (mx, my, right), (1 - mx, my, mz)):
            pl.semaphore_signal(
                barrier, inc=1,
                device_id=dev, device_id_type=pl.DeviceIdType.MESH,
            )

        @pl.when(has_yl)
        def _():
            pl.semaphore_signal(
                barrier, inc=1,
                device_id=(mx, my - 1, mz),
                device_id_type=pl.DeviceIdType.MESH,
            )

        @pl.when(has_yr)
        def _():
            pl.semaphore_signal(
                barrier, inc=1,
                device_id=(mx, my + 1, mz),
                device_id_type=pl.DeviceIdType.MESH,
            )

        pl.semaphore_wait(barrier, 3)
        @pl.when(has_yl)
        def _():
            pl.semaphore_wait(barrier, 1)

        @pl.when(has_yr)
        def _():
            pl.semaphore_wait(barrier, 1)

        def partial_into(dst, dst_row, c):
            x = o_ref[c]
            y = jnp.dot(x, w_ref[...], preferred_element_type=jnp.float32)
            dst[pl.ds(dst_row, PH), :] = y.astype(jnp.bfloat16)

        partial_into(out_ref, prow, (mz - 1) % N_Z)
        for s in range(N_Z - 1):
            rdma = pltpu.make_async_remote_copy(
                src_ref=out_ref.at[pl.ds(prow, PH)],
                dst_ref=comm_ref.at[s],
                send_sem=ring_send,
                recv_sem=ring_recv.at[s],
                device_id=(mx, my, right),
                device_id_type=pl.DeviceIdType.MESH,
            )
            rdma.start()
            partial_into(p_ref, 0, (mz - 2 - s) % N_Z)
            rdma.wait()
            out_ref[pl.ds(prow, PH), :] = p_ref[...] + comm_ref[s]

        orow0 = (1 - mx) * (M // 2)

        def x_send_desc(q, ss, rs):
            return pltpu.make_async_remote_copy(
                src_ref=out_ref.at[pl.ds(row0 + q * PH, PH)],
                dst_ref=out_ref.at[pl.ds(row0 + q * PH, PH)],
                send_sem=ss,
                recv_sem=rs,
                device_id=(1 - mx, my, mz),
                device_id_type=pl.DeviceIdType.MESH,
            )

        def x_recv_desc(q, ss, rs):
            return pltpu.make_async_remote_copy(
                src_ref=out_ref.at[pl.ds(orow0 + q * PH, PH)],
                dst_ref=out_ref.at[pl.ds(orow0 + q * PH, PH)],
                send_sem=ss,
                recv_sem=rs,
                device_id=(1 - mx, my, mz),
                device_id_type=pl.DeviceIdType.MESH,
            )

        def y_desc(t, q, dy, send_sems, recv_sems):
            return pltpu.make_async_remote_copy(
                src_ref=out_ref.at[pl.ds(row0 + q * PH, PH)],
                dst_ref=out_ref.at[pl.ds(row0 + q * PH, PH)],
                send_sem=send_sems.at[t],
                recv_sem=recv_sems.at[t],
                device_id=(mx, my + dy, mz),
                device_id_type=pl.DeviceIdType.MESH,
            )

        x_send_desc(my, xo_send, xo_recv).start()

        cond_r = [has_yl & (my - 1 - tt >= 0) for tt in range(N_Y - 1)]
        cond_l = [has_yr & (my + 1 + tt <= N_Y - 1) for tt in range(N_Y - 1)]

        for t in range(N_Y - 1):
            if t >= 1:
                @pl.when(cond_r[t - 1])
                def _():
                    y_desc(t - 1, my - t, 1, yr_send, yr_recv).wait_recv()
                    x_send_desc(my - t, xr_send.at[t - 1],
                                xr_recv.at[t - 1]).start()

                @pl.when(cond_l[t - 1])
                def _():
                    y_desc(t - 1, my + t, -1, yl_send, yl_recv).wait_recv()
                    x_send_desc(my + t, xl_send.at[t - 1],
                                xl_recv.at[t - 1]).start()

            @pl.when(has_yr & (my - t >= 0))
            def _():
                y_desc(t, my - t, 1, yr_send, yr_recv).start()

            @pl.when(has_yl & (my + t <= N_Y - 1))
            def _():
                y_desc(t, my + t, -1, yl_send, yl_recv).start()

        @pl.when(cond_r[N_Y - 2])
        def _():
            y_desc(N_Y - 2, my - (N_Y - 1), 1, yr_send, yr_recv).wait_recv()
            x_send_desc(my - (N_Y - 1), xr_send.at[N_Y - 2],
                        xr_recv.at[N_Y - 2]).start()

        @pl.when(cond_l[N_Y - 2])
        def _():
            y_desc(N_Y - 2, my + (N_Y - 1), -1, yl_send, yl_recv).wait_recv()
            x_send_desc(my + (N_Y - 1), xl_send.at[N_Y - 2],
                        xl_recv.at[N_Y - 2]).start()

        x_send_desc(my, xo_send, xo_recv).wait_send()
        x_recv_desc(my, xo_send, xo_recv).wait_recv()
        for tt in range(N_Y - 1):
            @pl.when(cond_r[tt])
            def _():
                q = my - 1 - tt
                x_send_desc(q, xr_send.at[tt], xr_recv.at[tt]).wait_send()
                x_recv_desc(q, xr_send.at[tt], xr_recv.at[tt]).wait_recv()

            @pl.when(cond_l[tt])
            def _():
                q = my + 1 + tt
                x_send_desc(q, xl_send.at[tt], xl_recv.at[tt]).wait_send()
                x_recv_desc(q, xl_send.at[tt], xl_recv.at[tt]).wait_recv()

        for t in range(N_Y - 1):
            @pl.when(has_yr & (my - t >= 0))
            def _():
                y_desc(t, my - t, 1, yr_send, yr_recv).wait_send()

            @pl.when(has_yl & (my + t <= N_Y - 1))
            def _():
                y_desc(t, my + t, -1, yl_send, yl_recv).wait_send()

        @functools.partial(pl.run_scoped, sem2=pltpu.SemaphoreType.REGULAR)
        def _(sem2):
            for dev in ((mx, my, left), (mx, my, right), (1 - mx, my, mz)):
                pl.semaphore_signal(
                    sem2, inc=1,
                    device_id=dev, device_id_type=pl.DeviceIdType.MESH,
                )

            @pl.when(has_yl)
            def _():
                pl.semaphore_signal(
                    sem2, inc=1,
                    device_id=(mx, my - 1, mz),
                    device_id_type=pl.DeviceIdType.MESH,
                )

            @pl.when(has_yr)
            def _():
                pl.semaphore_signal(
                    sem2, inc=1,
                    device_id=(mx, my + 1, mz),
                    device_id_type=pl.DeviceIdType.MESH,
                )

            pl.semaphore_wait(sem2, 3)
            @pl.when(has_yl)
            def _():
                pl.semaphore_wait(sem2, 1)

            @pl.when(has_yr)
            def _():
                pl.semaphore_wait(sem2, 1)

    out = pl.pallas_call(
        body,
        out_shape=jax.ShapeDtypeStruct((M, N), jnp.bfloat16),
        in_specs=[
            pl.BlockSpec(memory_space=pltpu.VMEM),
            pl.BlockSpec(memory_space=pltpu.VMEM),
        ],
        out_specs=pl.BlockSpec(memory_space=pltpu.VMEM),
        scratch_shapes=[
            pltpu.VMEM((N_Z - 1, PH, N), jnp.bfloat16),
            pltpu.VMEM((PH, N), jnp.bfloat16),
            pltpu.SemaphoreType.DMA,
            pltpu.SemaphoreType.DMA((N_Z - 1,)),
            pltpu.SemaphoreType.DMA((N_Y - 1,)),
            pltpu.SemaphoreType.DMA((N_Y - 1,)),
            pltpu.SemaphoreType.DMA((N_Y - 1,)),
            pltpu.SemaphoreType.DMA((N_Y - 1,)),
            pltpu.SemaphoreType.DMA,
            pltpu.SemaphoreType.DMA,
            pltpu.SemaphoreType.DMA((N_Y - 1,)),
            pltpu.SemaphoreType.DMA((N_Y - 1,)),
            pltpu.SemaphoreType.DMA((N_Y - 1,)),
            pltpu.SemaphoreType.DMA((N_Y - 1,)),
        ],
        compiler_params=pltpu.CompilerParams(
            collective_id=0,
            vmem_limit_bytes=100 * 1024 * 1024,
        ),
    )(Op, Wb)
    return out.reshape(B, Sc, N)
